# baseline (speedup 1.0000x reference)
import jax
import jax.numpy as jnp
from jax import lax
from jax.experimental import pallas as pl
from jax.experimental.pallas import tpu as pltpu

N = 4096
D_IN = 256
D_OUT = 64
NHEADS = 4
ALPHA = 0.2
BR = 1024
LOG2E = 1.4426950408889634
HSLOT = 128


def _gat_kernel(x_ref, adj_ref, wcat_ref, abd1_ref, abd2_ref,
                out_ref,
                whx_ref, e1_ref, e2t_ref):
    i = pl.program_id(0)

    @pl.when(i == 0)
    def _init():
        xb = x_ref[...].astype(jnp.bfloat16)
        wh_all = jnp.dot(xb, wcat_ref[...].astype(jnp.bfloat16),
                         preferred_element_type=jnp.float32)
        e1_ref[...] = (LOG2E * jnp.dot(
            wh_all, abd1_ref[...],
            preferred_element_type=jnp.float32)).astype(jnp.bfloat16)
        e2t_ref[...] = (LOG2E * lax.dot_general(
            abd2_ref[...], wh_all,
            (((0,), (1,)), ((), ())),
            preferred_element_type=jnp.float32)).astype(jnp.bfloat16)
        whb = wh_all.astype(jnp.bfloat16)
        ones_col = jnp.ones((N, 1), jnp.bfloat16)
        for h in range(NHEADS):
            whx_ref[:, h * HSLOT:h * HSLOT + D_OUT] = (
                whb[:, h * D_OUT:(h + 1) * D_OUT])
            whx_ref[:, h * HSLOT + D_OUT:h * HSLOT + D_OUT + 1] = ones_col

    madj = lax.bitcast_convert_type(
        adj_ref[...].astype(jnp.int16) * jnp.int16(0x3F80), jnp.bfloat16)
    e1_blk = e1_ref[pl.ds(i * BR, BR), :]
    for h in range(NHEADS):
        y = e1_blk[:, h:h + 1] + e2t_ref[h:h + 1, :]
        u = jnp.maximum(y, jnp.bfloat16(ALPHA) * y)
        p = jnp.exp2(u) * madj
        o = jnp.dot(p, whx_ref[:, h * HSLOT:(h + 1) * HSLOT],
                    preferred_element_type=jnp.float32)
        out_ref[:, h * D_OUT:(h + 1) * D_OUT] = (
            o[:, 0:D_OUT] * (1.0 / o[:, D_OUT:D_OUT + 1]))


def kernel(x, adj, W0, a0, W1, a1, W2, a2, W3, a3):
    Wcat = jnp.concatenate([W0, W1, W2, W3], axis=1)
    Abd1 = jax.scipy.linalg.block_diag(
        a0[:D_OUT], a1[:D_OUT], a2[:D_OUT], a3[:D_OUT])
    Abd2 = jax.scipy.linalg.block_diag(
        a0[D_OUT:], a1[D_OUT:], a2[D_OUT:], a3[D_OUT:])
    grid = (N // BR,)
    resident = lambda shape: pl.BlockSpec(shape, lambda i: (0, 0))
    out = pl.pallas_call(
        _gat_kernel,
        grid=grid,
        in_specs=[
            resident((N, D_IN)),
            pl.BlockSpec((BR, N), lambda i: (i, 0)),
            resident((D_IN, NHEADS * D_OUT)),
            resident((NHEADS * D_OUT, NHEADS)),
            resident((NHEADS * D_OUT, NHEADS)),
        ],
        out_specs=pl.BlockSpec((BR, NHEADS * D_OUT), lambda i: (i, 0)),
        out_shape=jax.ShapeDtypeStruct((N, NHEADS * D_OUT), jnp.float32),
        scratch_shapes=[
            pltpu.VMEM((N, NHEADS * HSLOT), jnp.bfloat16),
            pltpu.VMEM((N, NHEADS), jnp.bfloat16),
            pltpu.VMEM((NHEADS, N), jnp.bfloat16),
        ],
        compiler_params=pltpu.CompilerParams(
            dimension_semantics=("arbitrary",),
        ),
    )(x, adj, Wcat, Abd1, Abd2)
    return out


# speedup vs baseline: 1.0256x; 1.0256x over previous
import jax
import jax.numpy as jnp
from jax import lax
from jax.experimental import pallas as pl
from jax.experimental.pallas import tpu as pltpu

N = 4096
D_IN = 256
D_OUT = 64
NHEADS = 4
ALPHA = 0.2
BR = 512
LOG2E = 1.4426950408889634
HSLOT = 128


def _gat_kernel(x_ref, adj_ref, wcat_ref, abd1_ref, abd2_ref,
                out_ref,
                whx_ref, e1_ref, e2t_ref):
    i = pl.program_id(0)

    @pl.when(i == 0)
    def _init():
        xb = x_ref[...].astype(jnp.bfloat16)
        wh_all = jnp.dot(xb, wcat_ref[...].astype(jnp.bfloat16),
                         preferred_element_type=jnp.float32)
        e1_ref[...] = (LOG2E * jnp.dot(
            wh_all, abd1_ref[...],
            preferred_element_type=jnp.float32)).astype(jnp.bfloat16)
        e2t_ref[...] = (LOG2E * lax.dot_general(
            abd2_ref[...], wh_all,
            (((0,), (1,)), ((), ())),
            preferred_element_type=jnp.float32)).astype(jnp.bfloat16)
        whb = wh_all.astype(jnp.bfloat16)
        ones_col = jnp.ones((N, 1), jnp.bfloat16)
        for h in range(NHEADS):
            whx_ref[:, h * HSLOT:h * HSLOT + D_OUT] = (
                whb[:, h * D_OUT:(h + 1) * D_OUT])
            whx_ref[:, h * HSLOT + D_OUT:h * HSLOT + D_OUT + 1] = ones_col

    madj = lax.bitcast_convert_type(
        adj_ref[...].astype(jnp.int16) * jnp.int16(0x3F80), jnp.bfloat16)
    e1_blk = e1_ref[pl.ds(i * BR, BR), :]
    for h in range(NHEADS):
        y = e1_blk[:, h:h + 1] + e2t_ref[h:h + 1, :]
        u = jnp.maximum(y, jnp.bfloat16(ALPHA) * y)
        p = jnp.exp2(u) * madj
        o = jnp.dot(p, whx_ref[:, h * HSLOT:(h + 1) * HSLOT],
                    preferred_element_type=jnp.float32)
        out_ref[:, h * D_OUT:(h + 1) * D_OUT] = (
            o[:, 0:D_OUT] * (1.0 / o[:, D_OUT:D_OUT + 1]))


def kernel(x, adj, W0, a0, W1, a1, W2, a2, W3, a3):
    Wcat = jnp.concatenate([W0, W1, W2, W3], axis=1)
    Abd1 = jax.scipy.linalg.block_diag(
        a0[:D_OUT], a1[:D_OUT], a2[:D_OUT], a3[:D_OUT])
    Abd2 = jax.scipy.linalg.block_diag(
        a0[D_OUT:], a1[D_OUT:], a2[D_OUT:], a3[D_OUT:])
    grid = (N // BR,)
    resident = lambda shape: pl.BlockSpec(shape, lambda i: (0, 0))
    out = pl.pallas_call(
        _gat_kernel,
        grid=grid,
        in_specs=[
            resident((N, D_IN)),
            pl.BlockSpec((BR, N), lambda i: (i, 0)),
            resident((D_IN, NHEADS * D_OUT)),
            resident((NHEADS * D_OUT, NHEADS)),
            resident((NHEADS * D_OUT, NHEADS)),
        ],
        out_specs=pl.BlockSpec((BR, NHEADS * D_OUT), lambda i: (i, 0)),
        out_shape=jax.ShapeDtypeStruct((N, NHEADS * D_OUT), jnp.float32),
        scratch_shapes=[
            pltpu.VMEM((N, NHEADS * HSLOT), jnp.bfloat16),
            pltpu.VMEM((N, NHEADS), jnp.bfloat16),
            pltpu.VMEM((NHEADS, N), jnp.bfloat16),
        ],
        compiler_params=pltpu.CompilerParams(
            dimension_semantics=("arbitrary",),
        ),
    )(x, adj, Wcat, Abd1, Abd2)
    return out
